# baseline (speedup 1.0000x reference)
import functools

import jax
import jax.numpy as jnp
from jax.experimental import pallas as pl
from jax.experimental.pallas import tpu as pltpu

N = 10000
E = 160000
D = 256

EDGE_BLK = 2000
NODE_BLK = 1000

_JNP_NODE = False
_JNP_EDGE = False


def _bdot(a, b):
    return jax.lax.dot(a.astype(jnp.bfloat16), b.astype(jnp.bfloat16),
                       preferred_element_type=jnp.float32)


def _proj_body(x_ref, w_ref, os_ref, od_ref):
    p = _bdot(x_ref[...], w_ref[...])
    os_ref[...] = p[:, :D]
    od_ref[...] = p[:, D:]


def _project(x, w_sd):
    return pl.pallas_call(
        _proj_body,
        grid=(N // NODE_BLK,),
        in_specs=[
            pl.BlockSpec((NODE_BLK, D), lambda i: (i, 0)),
            pl.BlockSpec((D, 2 * D), lambda i: (0, 0)),
        ],
        out_specs=[
            pl.BlockSpec((NODE_BLK, D), lambda i: (i, 0)),
            pl.BlockSpec((NODE_BLK, D), lambda i: (i, 0)),
        ],
        out_shape=[
            jax.ShapeDtypeStruct((N, D), jnp.float32),
            jax.ShapeDtypeStruct((N, D), jnp.float32),
        ],
    )(x, w_sd)


def _edge_body(ea_ref, eb_ref, g_ref, w1a_ref, w2_ref, w3_ref,
               b1_ref, b2_ref, b3_ref, oa_ref, ob_ref):
    ea = ea_ref[...]
    eb = eb_ref[...]
    h = (_bdot(ea, w1a_ref[:D // 2, :])
         + _bdot(eb, w1a_ref[D // 2:, :])
         + g_ref[...] + b1_ref[...])
    h = jnp.maximum(
        _bdot(h, w2_ref[...]) + b2_ref[...], 0.0)
    h = _bdot(h, w3_ref[...]) + b3_ref[...]
    oa_ref[...] = ea + h[:, :D // 2]
    ob_ref[...] = eb + h[:, D // 2:]


def _edge_mlp(ea, eb, gsum, w1a, w2, w3, b1, b2, b3):
    wspec = pl.BlockSpec((D, D), lambda i: (0, 0))
    bspec = pl.BlockSpec((1, D), lambda i: (0, 0))
    hspec = pl.BlockSpec((EDGE_BLK, D // 2), lambda i: (i, 0))
    return pl.pallas_call(
        _edge_body,
        grid=(E // EDGE_BLK,),
        in_specs=[
            hspec, hspec,
            pl.BlockSpec((EDGE_BLK, D), lambda i: (i, 0)),
            wspec, wspec, wspec, bspec, bspec, bspec,
        ],
        out_specs=[hspec, hspec],
        out_shape=[
            jax.ShapeDtypeStruct((E, D // 2), jnp.float32),
            jax.ShapeDtypeStruct((E, D // 2), jnp.float32),
        ],
    )(ea, eb, gsum, w1a, w2, w3, b1, b2, b3)


def _node_body(x_ref, ps_ref, cnt_ref, wn1_ref, wn2_ref, wn3_ref, wn4_ref,
               bn1_ref, bn2_ref, bn3_ref, bn4_ref, o_ref):
    x = x_ref[...]
    s = jnp.concatenate([ps_ref[0], ps_ref[1]], axis=1)
    m = s * cnt_ref[:, :1]
    g = (_bdot(x, wn1_ref[:D, :])
         + _bdot(m, wn1_ref[D:2 * D, :])
         + _bdot(s, wn1_ref[2 * D:, :])
         + bn1_ref[...])
    g = jnp.maximum(g, 0.0)
    g = _bdot(g, wn2_ref[...]) + bn2_ref[...]
    g = jnp.maximum(_bdot(g, wn3_ref[...])
                    + bn3_ref[...], 0.0)
    g = jnp.maximum(_bdot(g, wn4_ref[...])
                    + bn4_ref[...], 0.0)
    o_ref[...] = x + g


def _node_mlp(x, pooled, cnt16, wn1, wn2, wn3, wn4, bn1, bn2, bn3, bn4):
    return pl.pallas_call(
        _node_body,
        grid=(N // NODE_BLK,),
        in_specs=[
            pl.BlockSpec((NODE_BLK, D), lambda i: (i, 0)),
            pl.BlockSpec((2, NODE_BLK, D // 2), lambda i: (0, i, 0)),
            pl.BlockSpec((NODE_BLK, 16), lambda i: (i, 0)),
            pl.BlockSpec((3 * D, 2 * D), lambda i: (0, 0)),
            pl.BlockSpec((2 * D, D), lambda i: (0, 0)),
            pl.BlockSpec((D, D), lambda i: (0, 0)),
            pl.BlockSpec((D, D), lambda i: (0, 0)),
            pl.BlockSpec((1, 2 * D), lambda i: (0, 0)),
            pl.BlockSpec((1, D), lambda i: (0, 0)),
            pl.BlockSpec((1, D), lambda i: (0, 0)),
            pl.BlockSpec((1, D), lambda i: (0, 0)),
        ],
        out_specs=pl.BlockSpec((NODE_BLK, D), lambda i: (i, 0)),
        out_shape=jax.ShapeDtypeStruct((N, D), jnp.float32),
    )(x, pooled, cnt16, wn1, wn2, wn3, wn4, bn1, bn2, bn3, bn4)


def kernel(x, edge_feat, edge_index, We1, be1, We2, be2, We3, be3,
           Wn1, bn1, Wn2, bn2, Wn3, bn3, Wn4, bn4):
    src = edge_index[0].astype(jnp.int32)
    dst = edge_index[1].astype(jnp.int32)

    w1a = We1[:D, :]
    w_sd = We1[D:, :].reshape(2, D, D)
    w_sd = jnp.concatenate([w_sd[0], w_sd[1]], axis=1)

    b1 = be1.reshape(1, D)
    b2 = be2.reshape(1, D)
    b3 = be3.reshape(1, D)
    nb1 = bn1.reshape(1, 2 * D)
    nb2 = bn2.reshape(1, D)
    nb3 = bn3.reshape(1, D)
    nb4 = bn4.reshape(1, D)

    ea = edge_feat[:, :D // 2]
    eb = edge_feat[:, D // 2:]

    cnt = jax.ops.segment_sum(jnp.ones((E,), jnp.float32), dst, num_segments=N)
    cnt16 = jnp.broadcast_to((1.0 / jnp.maximum(cnt, 1.0))[:, None], (N, 16))

    for _ in range(3):
        xs, xd = _project(x, w_sd)
        gsum = jnp.take(xs, src, axis=0) + jnp.take(xd, dst, axis=0)
        if _JNP_EDGE:
            h = ea @ w1a[:D // 2] + eb @ w1a[D // 2:] + gsum + be1
            h = jax.nn.relu(h @ We2 + be2)
            h = h @ We3 + be3
            ea = ea + h[:, :D // 2]
            eb = eb + h[:, D // 2:]
        else:
            ea, eb = _edge_mlp(ea, eb, gsum, w1a, We2, We3, b1, b2, b3)
        pa = jax.ops.segment_sum(ea, dst, num_segments=N)
        pb = jax.ops.segment_sum(eb, dst, num_segments=N)
        pooled = jnp.stack([pa, pb])
        if _JNP_NODE:
            s = jnp.concatenate([pa, pb], axis=1)
            m = s * cnt16[:, :1]
            g = jax.nn.relu(x @ Wn1[:D] + m @ Wn1[D:2 * D] + s @ Wn1[2 * D:] + bn1)
            g = g @ Wn2 + bn2
            g = jax.nn.relu(g @ Wn3 + bn3)
            g = jax.nn.relu(g @ Wn4 + bn4)
            x = x + g
        else:
            x = _node_mlp(x, pooled, cnt16, Wn1, Wn2, Wn3, Wn4, nb1, nb2, nb3, nb4)

    e = jnp.concatenate([ea, eb], axis=1)
    return (x, e)


# speedup vs baseline: 2.6384x; 2.6384x over previous
import functools

import jax
import jax.numpy as jnp
from jax import lax
from jax.experimental import pallas as pl
from jax.experimental.pallas import tpu as pltpu
from jax.experimental.pallas import tpu_sc as plsc

N = 10000
E = 160000
D = 256
H = D // 2

EDGE_BLK = 2000
NODE_BLK = 1000

CH = 128
NCHUNK = E // CH
TROWS = 640
NPAD = 16 * TROWS

_MESH = plsc.VectorSubcoreMesh(core_axis_name="c", subcore_axis_name="s",
                               num_cores=2, num_subcores=16)


def _bdot(a, b):
    return jax.lax.dot(a.astype(jnp.bfloat16), b.astype(jnp.bfloat16),
                       preferred_element_type=jnp.float32)


def _proj_body(x_ref, w_ref, os_ref, od_ref):
    p = _bdot(x_ref[...], w_ref[...])
    os_ref[...] = p[:, :D]
    od_ref[...] = p[:, D:]


def _project(x, w_sd):
    return pl.pallas_call(
        _proj_body,
        grid=(N // NODE_BLK,),
        in_specs=[
            pl.BlockSpec((NODE_BLK, D), lambda i: (i, 0)),
            pl.BlockSpec((D, 2 * D), lambda i: (0, 0)),
        ],
        out_specs=[
            pl.BlockSpec((NODE_BLK, D), lambda i: (i, 0)),
            pl.BlockSpec((NODE_BLK, D), lambda i: (i, 0)),
        ],
        out_shape=[
            jax.ShapeDtypeStruct((N, D), jnp.float32),
            jax.ShapeDtypeStruct((N, D), jnp.float32),
        ],
    )(x, w_sd)


def _edge_body(e_ref, g_ref, g2_ref, w1_ref, w2_ref, w3_ref,
               b1_ref, b2_ref, b3_ref, o_ref):
    e = e_ref[...]
    h = _bdot(e, w1_ref[...]) + (g_ref[...] + g2_ref[...]) + b1_ref[...]
    h = jnp.maximum(_bdot(h, w2_ref[...]) + b2_ref[...], 0.0)
    h = _bdot(h, w3_ref[...]) + b3_ref[...]
    o_ref[...] = e + h


def _edge_mlp(e, gs, gd, w1a, w2, w3, b1, b2, b3):
    wspec = pl.BlockSpec((D, D), lambda i: (0, 0))
    bspec = pl.BlockSpec((1, D), lambda i: (0, 0))
    espec = pl.BlockSpec((EDGE_BLK, D), lambda i: (i, 0))
    return pl.pallas_call(
        _edge_body,
        grid=(E // EDGE_BLK,),
        in_specs=[espec, espec, espec, wspec, wspec, wspec, bspec, bspec, bspec],
        out_specs=espec,
        out_shape=jax.ShapeDtypeStruct((E, D), jnp.float32),
    )(e, gs, gd, w1a, w2, w3, b1, b2, b3)


def _node_body(x_ref, ps_ref, rec_ref, wn1_ref, wn2_ref, wn3_ref, wn4_ref,
               bn1_ref, bn2_ref, bn3_ref, bn4_ref, o_ref):
    x = x_ref[...]
    s = jnp.concatenate([ps_ref[0], ps_ref[1]], axis=1)
    m = s * rec_ref[:, :1]
    g = (_bdot(x, wn1_ref[:D, :]) + _bdot(m, wn1_ref[D:2 * D, :])
         + _bdot(s, wn1_ref[2 * D:, :]) + bn1_ref[...])
    g = jnp.maximum(g, 0.0)
    g = _bdot(g, wn2_ref[...]) + bn2_ref[...]
    g = jnp.maximum(_bdot(g, wn3_ref[...]) + bn3_ref[...], 0.0)
    g = jnp.maximum(_bdot(g, wn4_ref[...]) + bn4_ref[...], 0.0)
    o_ref[...] = x + g


def _node_mlp(x, pooled, rec16, wn1, wn2, wn3, wn4, bn1, bn2, bn3, bn4):
    return pl.pallas_call(
        _node_body,
        grid=(N // NODE_BLK,),
        in_specs=[
            pl.BlockSpec((NODE_BLK, D), lambda i: (i, 0)),
            pl.BlockSpec((2, NODE_BLK, H), lambda i: (0, i, 0)),
            pl.BlockSpec((NODE_BLK, 16), lambda i: (i, 0)),
            pl.BlockSpec((3 * D, 2 * D), lambda i: (0, 0)),
            pl.BlockSpec((2 * D, D), lambda i: (0, 0)),
            pl.BlockSpec((D, D), lambda i: (0, 0)),
            pl.BlockSpec((D, D), lambda i: (0, 0)),
            pl.BlockSpec((1, 2 * D), lambda i: (0, 0)),
            pl.BlockSpec((1, D), lambda i: (0, 0)),
            pl.BlockSpec((1, D), lambda i: (0, 0)),
            pl.BlockSpec((1, D), lambda i: (0, 0)),
        ],
        out_specs=pl.BlockSpec((NODE_BLK, D), lambda i: (i, 0)),
        out_shape=jax.ShapeDtypeStruct((N, D), jnp.float32),
    )(x, pooled, rec16, wn1, wn2, wn3, wn4, bn1, bn2, bn3, bn4)


def _gather_body(xs_hbm, xd_hbm, src_hbm, dst_hbm, gs_hbm, gd_hbm,
                 idx_s, idx_d, buf, sem):
    c = lax.axis_index("c")
    s = lax.axis_index("s")
    wid = s * 2 + c
    nfull = NCHUNK // 32
    nch = jnp.where(wid < NCHUNK - 32 * nfull, nfull + 1, nfull)

    def step(j, carry):
        base = (wid + 32 * j) * CH
        pltpu.sync_copy(src_hbm.at[pl.ds(base, CH)], idx_s)
        pltpu.sync_copy(dst_hbm.at[pl.ds(base, CH)], idx_d)
        pltpu.async_copy(xs_hbm.at[idx_s], buf, sem).wait()
        pltpu.sync_copy(buf, gs_hbm.at[pl.ds(base, CH), :])
        pltpu.async_copy(xd_hbm.at[idx_d], buf, sem).wait()
        pltpu.sync_copy(buf, gd_hbm.at[pl.ds(base, CH), :])
        return carry

    lax.fori_loop(0, nch, step, 0)


_sc_gather = functools.partial(
    pl.kernel,
    out_type=[
        jax.ShapeDtypeStruct((E, D), jnp.float32),
        jax.ShapeDtypeStruct((E, D), jnp.float32),
    ],
    mesh=_MESH,
    scratch_types=[
        pltpu.VMEM((CH,), jnp.int32),
        pltpu.VMEM((CH,), jnp.int32),
        pltpu.VMEM((CH, D), jnp.float32),
        pltpu.SemaphoreType.DMA,
    ],
)(_gather_body)


def _zero_fill(ref, rows, width):
    def st(t, carry):
        r = t // (width // 16)
        q = t % (width // 16)
        ref[r, pl.ds(q * 16, 16)] = jnp.zeros((16,), jnp.float32)
        return carry
    lax.fori_loop(0, rows * (width // 16), st, 0)


def _scatter_common(e_hbm, dst_hbm, out_hbm, acc, idx_v, buf, sem,
                    cnt_extra=None):
    c = lax.axis_index("c")
    s = lax.axis_index("s")

    _zero_fill(buf, CH, H)
    for q in range(TROWS // CH):
        pltpu.sync_copy(buf, acc.at[pl.ds(s * TROWS + q * CH, CH)])
    if cnt_extra is not None:
        cacc, ones = cnt_extra

        @pl.when(c == 0)
        def _():
            _zero_fill(ones, CH, 16)
            for q in range(TROWS // CH):
                pltpu.sync_copy(ones,
                                cacc.at[pl.ds(s * TROWS + q * CH, CH)])

            def fill_ones(t, carry):
                ones[t, :] = jnp.full((16,), 1.0, jnp.float32)
                return carry
            lax.fori_loop(0, CH, fill_ones, 0)
    plsc.subcore_barrier()

    nfull = NCHUNK // 16
    nch = jnp.where(s < NCHUNK - 16 * nfull, nfull + 1, nfull)

    def step(j, carry):
        base = (s + 16 * j) * CH
        pltpu.sync_copy(dst_hbm.at[pl.ds(base, CH)], idx_v)

        @pl.when(c == 0)
        def _():
            pltpu.sync_copy(e_hbm.at[pl.ds(base, CH), pl.ds(0, H)], buf)

        @pl.when(c == 1)
        def _():
            pltpu.sync_copy(e_hbm.at[pl.ds(base, CH), pl.ds(H, H)], buf)

        pltpu.async_copy(buf, acc.at[idx_v], sem, add=True).wait()
        if cnt_extra is not None:
            cacc, ones = cnt_extra

            @pl.when(c == 0)
            def _():
                pltpu.async_copy(ones, cacc.at[idx_v], sem, add=True).wait()
        return carry

    lax.fori_loop(0, nch, step, 0)
    plsc.subcore_barrier()

    @pl.when(s < 15)
    def _():
        for q in range(TROWS // CH):
            r0 = s * TROWS + q * CH
            pltpu.sync_copy(acc.at[pl.ds(r0, CH)], buf)
            pltpu.sync_copy(buf, out_hbm.at[c, pl.ds(r0, CH), :])

    @pl.when(s == 15)
    def _():
        for q in range(3):
            r0 = 15 * TROWS + q * CH
            pltpu.sync_copy(acc.at[pl.ds(r0, CH)], buf)
            pltpu.sync_copy(buf, out_hbm.at[c, pl.ds(r0, CH), :])
        r0 = 15 * TROWS + 3 * CH
        pltpu.sync_copy(acc.at[pl.ds(r0, 16)], buf.at[pl.ds(0, 16)])
        pltpu.sync_copy(buf.at[pl.ds(0, 16)], out_hbm.at[c, pl.ds(r0, 16), :])


def _scatter_body(e_hbm, dst_hbm, out_hbm, acc, idx_v, buf, sem):
    _scatter_common(e_hbm, dst_hbm, out_hbm, acc, idx_v, buf, sem)


def _scatter_cnt_body(e_hbm, dst_hbm, out_hbm, cnt_hbm,
                      acc, cacc, idx_v, buf, ones, sem):
    c = lax.axis_index("c")
    s = lax.axis_index("s")
    _scatter_common(e_hbm, dst_hbm, out_hbm, acc, idx_v, buf, sem,
                    cnt_extra=(cacc, ones))

    @pl.when((c == 0) & (s < 15))
    def _():
        for q in range(TROWS // CH):
            r0 = s * TROWS + q * CH
            pltpu.sync_copy(cacc.at[pl.ds(r0, CH)], ones)
            pltpu.sync_copy(ones, cnt_hbm.at[pl.ds(r0, CH), :])

    @pl.when((c == 0) & (s == 15))
    def _():
        for q in range(3):
            r0 = 15 * TROWS + q * CH
            pltpu.sync_copy(cacc.at[pl.ds(r0, CH)], ones)
            pltpu.sync_copy(ones, cnt_hbm.at[pl.ds(r0, CH), :])
        r0 = 15 * TROWS + 3 * CH
        pltpu.sync_copy(cacc.at[pl.ds(r0, 16)], ones.at[pl.ds(0, 16)])
        pltpu.sync_copy(ones.at[pl.ds(0, 16)], cnt_hbm.at[pl.ds(r0, 16), :])


_sc_scatter = functools.partial(
    pl.kernel,
    out_type=jax.ShapeDtypeStruct((2, N, H), jnp.float32),
    mesh=_MESH,
    scratch_types=[
        pltpu.VMEM_SHARED((NPAD, H), jnp.float32),
        pltpu.VMEM((CH,), jnp.int32),
        pltpu.VMEM((CH, H), jnp.float32),
        pltpu.SemaphoreType.DMA,
    ],
)(_scatter_body)


_sc_scatter_cnt = functools.partial(
    pl.kernel,
    out_type=[
        jax.ShapeDtypeStruct((2, N, H), jnp.float32),
        jax.ShapeDtypeStruct((N, 16), jnp.float32),
    ],
    mesh=_MESH,
    scratch_types=[
        pltpu.VMEM_SHARED((NPAD, H), jnp.float32),
        pltpu.VMEM_SHARED((NPAD, 16), jnp.float32),
        pltpu.VMEM((CH,), jnp.int32),
        pltpu.VMEM((CH, H), jnp.float32),
        pltpu.VMEM((CH, 16), jnp.float32),
        pltpu.SemaphoreType.DMA,
    ],
)(_scatter_cnt_body)


def kernel(x, edge_feat, edge_index, We1, be1, We2, be2, We3, be3,
           Wn1, bn1, Wn2, bn2, Wn3, bn3, Wn4, bn4):
    src32 = edge_index[0].astype(jnp.int32)
    dst32 = edge_index[1].astype(jnp.int32)

    w1a = We1[:D, :]
    w_sd = We1[D:, :].reshape(2, D, D)
    w_sd = jnp.concatenate([w_sd[0], w_sd[1]], axis=1)

    b1 = be1.reshape(1, D)
    b2 = be2.reshape(1, D)
    b3 = be3.reshape(1, D)
    nb1 = bn1.reshape(1, 2 * D)
    nb2 = bn2.reshape(1, D)
    nb3 = bn3.reshape(1, D)
    nb4 = bn4.reshape(1, D)

    e = edge_feat
    rec16 = None
    for it in range(3):
        xs, xd = _project(x, w_sd)
        gs, gd = _sc_gather(xs, xd, src32, dst32)
        e = _edge_mlp(e, gs, gd, w1a, We2, We3, b1, b2, b3)
        if it == 0:
            cnt = jax.ops.segment_sum(jnp.ones((E,), jnp.float32), dst32,
                                      num_segments=N)
            rec16 = jnp.broadcast_to(
                (1.0 / jnp.maximum(cnt, 1.0))[:, None], (N, 16))
        pooled = _sc_scatter(e, dst32)
        x = _node_mlp(x, pooled, rec16, Wn1, Wn2, Wn3, Wn4, nb1, nb2, nb3, nb4)

    return (x, e)


# speedup vs baseline: 2.6730x; 1.0131x over previous
import functools

import jax
import jax.numpy as jnp
from jax import lax
from jax.experimental import pallas as pl
from jax.experimental.pallas import tpu as pltpu
from jax.experimental.pallas import tpu_sc as plsc

N = 10000
E = 160000
D = 256
H = D // 2

EDGE_BLK = 2000
NODE_BLK = 1000

CH = 128
NCHUNK = E // CH
TROWS = 640
NPAD = 16 * TROWS

_MESH = plsc.VectorSubcoreMesh(core_axis_name="c", subcore_axis_name="s",
                               num_cores=2, num_subcores=16)


def _bdot(a, b):
    return jax.lax.dot(a.astype(jnp.bfloat16), b.astype(jnp.bfloat16),
                       preferred_element_type=jnp.float32)


def _proj_body(x_ref, w_ref, os_ref, od_ref):
    p = _bdot(x_ref[...], w_ref[...])
    os_ref[...] = p[:, :D]
    od_ref[...] = p[:, D:]


def _project(x, w_sd):
    return pl.pallas_call(
        _proj_body,
        grid=(N // NODE_BLK,),
        in_specs=[
            pl.BlockSpec((NODE_BLK, D), lambda i: (i, 0)),
            pl.BlockSpec((D, 2 * D), lambda i: (0, 0)),
        ],
        out_specs=[
            pl.BlockSpec((NODE_BLK, D), lambda i: (i, 0)),
            pl.BlockSpec((NODE_BLK, D), lambda i: (i, 0)),
        ],
        out_shape=[
            jax.ShapeDtypeStruct((N, D), jnp.float32),
            jax.ShapeDtypeStruct((N, D), jnp.float32),
        ],
    )(x, w_sd)


def _edge_body(e_ref, g_ref, g2_ref, w1_ref, w2_ref, w3_ref,
               b1_ref, b2_ref, b3_ref, o_ref):
    e = e_ref[...]
    h = _bdot(e, w1_ref[...]) + (g_ref[...] + g2_ref[...]) + b1_ref[...]
    h = jnp.maximum(_bdot(h, w2_ref[...]) + b2_ref[...], 0.0)
    h = _bdot(h, w3_ref[...]) + b3_ref[...]
    o_ref[...] = e + h


def _edge_mlp(e, gs, gd, w1a, w2, w3, b1, b2, b3):
    wspec = pl.BlockSpec((D, D), lambda i: (0, 0))
    bspec = pl.BlockSpec((1, D), lambda i: (0, 0))
    espec = pl.BlockSpec((EDGE_BLK, D), lambda i: (i, 0))
    return pl.pallas_call(
        _edge_body,
        grid=(E // EDGE_BLK,),
        in_specs=[espec, espec, espec, wspec, wspec, wspec, bspec, bspec, bspec],
        out_specs=espec,
        out_shape=jax.ShapeDtypeStruct((E, D), jnp.float32),
    )(e, gs, gd, w1a, w2, w3, b1, b2, b3)


def _node_body(x_ref, ps_ref, rec_ref, wn1_ref, wn2_ref, wn3_ref, wn4_ref,
               bn1_ref, bn2_ref, bn3_ref, bn4_ref, o_ref):
    x = x_ref[...]
    s = jnp.concatenate([ps_ref[0], ps_ref[1]], axis=1)
    m = s * rec_ref[:, :1]
    g = (_bdot(x, wn1_ref[:D, :]) + _bdot(m, wn1_ref[D:2 * D, :])
         + _bdot(s, wn1_ref[2 * D:, :]) + bn1_ref[...])
    g = jnp.maximum(g, 0.0)
    g = _bdot(g, wn2_ref[...]) + bn2_ref[...]
    g = jnp.maximum(_bdot(g, wn3_ref[...]) + bn3_ref[...], 0.0)
    g = jnp.maximum(_bdot(g, wn4_ref[...]) + bn4_ref[...], 0.0)
    o_ref[...] = x + g


def _node_mlp(x, pooled, rec16, wn1, wn2, wn3, wn4, bn1, bn2, bn3, bn4):
    return pl.pallas_call(
        _node_body,
        grid=(N // NODE_BLK,),
        in_specs=[
            pl.BlockSpec((NODE_BLK, D), lambda i: (i, 0)),
            pl.BlockSpec((2, NODE_BLK, H), lambda i: (0, i, 0)),
            pl.BlockSpec((NODE_BLK, 16), lambda i: (i, 0)),
            pl.BlockSpec((3 * D, 2 * D), lambda i: (0, 0)),
            pl.BlockSpec((2 * D, D), lambda i: (0, 0)),
            pl.BlockSpec((D, D), lambda i: (0, 0)),
            pl.BlockSpec((D, D), lambda i: (0, 0)),
            pl.BlockSpec((1, 2 * D), lambda i: (0, 0)),
            pl.BlockSpec((1, D), lambda i: (0, 0)),
            pl.BlockSpec((1, D), lambda i: (0, 0)),
            pl.BlockSpec((1, D), lambda i: (0, 0)),
        ],
        out_specs=pl.BlockSpec((NODE_BLK, D), lambda i: (i, 0)),
        out_shape=jax.ShapeDtypeStruct((N, D), jnp.float32),
    )(x, pooled, rec16, wn1, wn2, wn3, wn4, bn1, bn2, bn3, bn4)


def _gather_body(xs_hbm, xd_hbm, src_hbm, dst_hbm, gs_hbm, gd_hbm,
                 idx_s, idx_d, buf, sem):
    c = lax.axis_index("c")
    s = lax.axis_index("s")
    wid = s * 2 + c
    nfull = NCHUNK // 32
    nch = jnp.where(wid < NCHUNK - 32 * nfull, nfull + 1, nfull)

    def step(j, carry):
        base = (wid + 32 * j) * CH
        pltpu.sync_copy(src_hbm.at[pl.ds(base, CH)], idx_s)
        pltpu.sync_copy(dst_hbm.at[pl.ds(base, CH)], idx_d)
        pltpu.async_copy(xs_hbm.at[idx_s], buf, sem).wait()
        pltpu.sync_copy(buf, gs_hbm.at[pl.ds(base, CH), :])
        pltpu.async_copy(xd_hbm.at[idx_d], buf, sem).wait()
        pltpu.sync_copy(buf, gd_hbm.at[pl.ds(base, CH), :])
        return carry

    lax.fori_loop(0, nch, step, 0)


_sc_gather = functools.partial(
    pl.kernel,
    out_type=[
        jax.ShapeDtypeStruct((E, D), jnp.float32),
        jax.ShapeDtypeStruct((E, D), jnp.float32),
    ],
    mesh=_MESH,
    scratch_types=[
        pltpu.VMEM((CH,), jnp.int32),
        pltpu.VMEM((CH,), jnp.int32),
        pltpu.VMEM((CH, D), jnp.float32),
        pltpu.SemaphoreType.DMA,
    ],
)(_gather_body)


def _zero_fill(ref, rows, width):
    def st(t, carry):
        r = t // (width // 16)
        q = t % (width // 16)
        ref[r, pl.ds(q * 16, 16)] = jnp.zeros((16,), jnp.float32)
        return carry
    lax.fori_loop(0, rows * (width // 16), st, 0)


def _scatter_common(e_hbm, dst_hbm, out_hbm, acc, idx_v, buf, sem,
                    cnt_extra=None):
    c = lax.axis_index("c")
    s = lax.axis_index("s")

    _zero_fill(buf, CH, H)
    for q in range(TROWS // CH):
        pltpu.sync_copy(buf, acc.at[pl.ds(s * TROWS + q * CH, CH)])
    if cnt_extra is not None:
        cacc, ones = cnt_extra

        @pl.when(c == 0)
        def _():
            _zero_fill(ones, CH, 16)
            for q in range(TROWS // CH):
                pltpu.sync_copy(ones,
                                cacc.at[pl.ds(s * TROWS + q * CH, CH)])

            def fill_ones(t, carry):
                ones[t, :] = jnp.full((16,), 1.0, jnp.float32)
                return carry
            lax.fori_loop(0, CH, fill_ones, 0)
    plsc.subcore_barrier()

    nfull = NCHUNK // 16
    nch = jnp.where(s < NCHUNK - 16 * nfull, nfull + 1, nfull)

    def step(j, carry):
        base = (s + 16 * j) * CH
        pltpu.sync_copy(dst_hbm.at[pl.ds(base, CH)], idx_v)

        @pl.when(c == 0)
        def _():
            pltpu.sync_copy(e_hbm.at[pl.ds(base, CH), pl.ds(0, H)], buf)

        @pl.when(c == 1)
        def _():
            pltpu.sync_copy(e_hbm.at[pl.ds(base, CH), pl.ds(H, H)], buf)

        pltpu.async_copy(buf, acc.at[idx_v], sem, add=True).wait()
        if cnt_extra is not None:
            cacc, ones = cnt_extra

            @pl.when(c == 0)
            def _():
                pltpu.async_copy(ones, cacc.at[idx_v], sem, add=True).wait()
        return carry

    lax.fori_loop(0, nch, step, 0)
    plsc.subcore_barrier()

    @pl.when(s < 15)
    def _():
        for q in range(TROWS // CH):
            r0 = s * TROWS + q * CH
            pltpu.sync_copy(acc.at[pl.ds(r0, CH)], buf)
            pltpu.sync_copy(buf, out_hbm.at[c, pl.ds(r0, CH), :])

    @pl.when(s == 15)
    def _():
        for q in range(3):
            r0 = 15 * TROWS + q * CH
            pltpu.sync_copy(acc.at[pl.ds(r0, CH)], buf)
            pltpu.sync_copy(buf, out_hbm.at[c, pl.ds(r0, CH), :])
        r0 = 15 * TROWS + 3 * CH
        pltpu.sync_copy(acc.at[pl.ds(r0, 16)], buf.at[pl.ds(0, 16)])
        pltpu.sync_copy(buf.at[pl.ds(0, 16)], out_hbm.at[c, pl.ds(r0, 16), :])


def _scatter_body(e_hbm, dst_hbm, out_hbm, acc, idx_v, buf, sem):
    _scatter_common(e_hbm, dst_hbm, out_hbm, acc, idx_v, buf, sem)


def _scatter_cnt_body(e_hbm, dst_hbm, out_hbm, cnt_hbm,
                      acc, cacc, idx_v, buf, ones, sem):
    c = lax.axis_index("c")
    s = lax.axis_index("s")
    _scatter_common(e_hbm, dst_hbm, out_hbm, acc, idx_v, buf, sem,
                    cnt_extra=(cacc, ones))

    @pl.when((c == 0) & (s < 15))
    def _():
        for q in range(TROWS // CH):
            r0 = s * TROWS + q * CH
            pltpu.sync_copy(cacc.at[pl.ds(r0, CH)], ones)
            pltpu.sync_copy(ones, cnt_hbm.at[pl.ds(r0, CH), :])

    @pl.when((c == 0) & (s == 15))
    def _():
        for q in range(3):
            r0 = 15 * TROWS + q * CH
            pltpu.sync_copy(cacc.at[pl.ds(r0, CH)], ones)
            pltpu.sync_copy(ones, cnt_hbm.at[pl.ds(r0, CH), :])
        r0 = 15 * TROWS + 3 * CH
        pltpu.sync_copy(cacc.at[pl.ds(r0, 16)], ones.at[pl.ds(0, 16)])
        pltpu.sync_copy(ones.at[pl.ds(0, 16)], cnt_hbm.at[pl.ds(r0, 16), :])


def _count_body(dst_hbm, out_hbm, acc, idx_v, buf, sem):
    c = lax.axis_index("c")
    s = lax.axis_index("s")

    _zero_fill(buf, CH, H)
    for q in range(TROWS // CH):
        pltpu.sync_copy(buf, acc.at[pl.ds(s * TROWS + q * CH, CH)])

    def fill_ones(t, carry):
        for q in range(H // 16):
            buf[t, pl.ds(q * 16, 16)] = jnp.full((16,), 1.0, jnp.float32)
        return carry
    lax.fori_loop(0, CH, fill_ones, 0)
    plsc.subcore_barrier()

    nfull = NCHUNK // 32
    wid = s * 2 + c
    nch = jnp.where(wid < NCHUNK - 32 * nfull, nfull + 1, nfull)

    def step(j, carry):
        base = (wid + 32 * j) * CH
        pltpu.sync_copy(dst_hbm.at[pl.ds(base, CH)], idx_v)
        pltpu.async_copy(buf, acc.at[idx_v], sem, add=True).wait()
        return carry

    lax.fori_loop(0, nch, step, 0)
    plsc.subcore_barrier()

    @pl.when(s < 15)
    def _():
        for q in range(TROWS // CH):
            r0 = s * TROWS + q * CH
            pltpu.sync_copy(acc.at[pl.ds(r0, CH)], buf)
            pltpu.sync_copy(buf, out_hbm.at[c, pl.ds(r0, CH), :])

    @pl.when(s == 15)
    def _():
        for q in range(3):
            r0 = 15 * TROWS + q * CH
            pltpu.sync_copy(acc.at[pl.ds(r0, CH)], buf)
            pltpu.sync_copy(buf, out_hbm.at[c, pl.ds(r0, CH), :])
        r0 = 15 * TROWS + 3 * CH
        pltpu.sync_copy(acc.at[pl.ds(r0, 16)], buf.at[pl.ds(0, 16)])
        pltpu.sync_copy(buf.at[pl.ds(0, 16)], out_hbm.at[c, pl.ds(r0, 16), :])


_sc_count = functools.partial(
    pl.kernel,
    out_type=jax.ShapeDtypeStruct((2, N, H), jnp.float32),
    mesh=_MESH,
    scratch_types=[
        pltpu.VMEM_SHARED((NPAD, H), jnp.float32),
        pltpu.VMEM((CH,), jnp.int32),
        pltpu.VMEM((CH, H), jnp.float32),
        pltpu.SemaphoreType.DMA,
    ],
)(_count_body)


_sc_scatter = functools.partial(
    pl.kernel,
    out_type=jax.ShapeDtypeStruct((2, N, H), jnp.float32),
    mesh=_MESH,
    scratch_types=[
        pltpu.VMEM_SHARED((NPAD, H), jnp.float32),
        pltpu.VMEM((CH,), jnp.int32),
        pltpu.VMEM((CH, H), jnp.float32),
        pltpu.SemaphoreType.DMA,
    ],
)(_scatter_body)


_sc_scatter_cnt = functools.partial(
    pl.kernel,
    out_type=[
        jax.ShapeDtypeStruct((2, N, H), jnp.float32),
        jax.ShapeDtypeStruct((N, 16), jnp.float32),
    ],
    mesh=_MESH,
    scratch_types=[
        pltpu.VMEM_SHARED((NPAD, H), jnp.float32),
        pltpu.VMEM_SHARED((NPAD, 16), jnp.float32),
        pltpu.VMEM((CH,), jnp.int32),
        pltpu.VMEM((CH, H), jnp.float32),
        pltpu.VMEM((CH, 16), jnp.float32),
        pltpu.SemaphoreType.DMA,
    ],
)(_scatter_cnt_body)


def kernel(x, edge_feat, edge_index, We1, be1, We2, be2, We3, be3,
           Wn1, bn1, Wn2, bn2, Wn3, bn3, Wn4, bn4):
    src32 = edge_index[0].astype(jnp.int32)
    dst32 = edge_index[1].astype(jnp.int32)

    w1a = We1[:D, :]
    w_sd = We1[D:, :].reshape(2, D, D)
    w_sd = jnp.concatenate([w_sd[0], w_sd[1]], axis=1)

    b1 = be1.reshape(1, D)
    b2 = be2.reshape(1, D)
    b3 = be3.reshape(1, D)
    nb1 = bn1.reshape(1, 2 * D)
    nb2 = bn2.reshape(1, D)
    nb3 = bn3.reshape(1, D)
    nb4 = bn4.reshape(1, D)

    e = edge_feat
    rec16 = None
    for it in range(3):
        xs, xd = _project(x, w_sd)
        gs, gd = _sc_gather(xs, xd, src32, dst32)
        e = _edge_mlp(e, gs, gd, w1a, We2, We3, b1, b2, b3)
        if it == 0:
            cnt2 = _sc_count(dst32)
            cnt16 = cnt2[0, :, :16] + cnt2[1, :, :16]
            rec16 = 1.0 / jnp.maximum(cnt16, 1.0)
        pooled = _sc_scatter(e, dst32)
        x = _node_mlp(x, pooled, rec16, Wn1, Wn2, Wn3, Wn4, nb1, nb2, nb3, nb4)

    return (x, e)


# speedup vs baseline: 2.8040x; 1.0490x over previous
import functools

import jax
import jax.numpy as jnp
from jax import lax
from jax.experimental import pallas as pl
from jax.experimental.pallas import tpu as pltpu
from jax.experimental.pallas import tpu_sc as plsc

N = 10000
E = 160000
D = 256
H = D // 2

EDGE_BLK = 2000
NODE_BLK = 1000

CH = 128
NCHUNK = E // CH
TROWS = 640
NPAD = 16 * TROWS

_MESH = plsc.VectorSubcoreMesh(core_axis_name="c", subcore_axis_name="s",
                               num_cores=2, num_subcores=16)


def _bdot(a, b):
    return jax.lax.dot(a.astype(jnp.bfloat16), b.astype(jnp.bfloat16),
                       preferred_element_type=jnp.float32)


def _proj_body(x_ref, w_ref, os_ref, od_ref):
    p = _bdot(x_ref[...], w_ref[...])
    os_ref[...] = p[:, :D]
    od_ref[...] = p[:, D:]


def _project(x, w_sd):
    return pl.pallas_call(
        _proj_body,
        grid=(N // NODE_BLK,),
        in_specs=[
            pl.BlockSpec((NODE_BLK, D), lambda i: (i, 0)),
            pl.BlockSpec((D, 2 * D), lambda i: (0, 0)),
        ],
        out_specs=[
            pl.BlockSpec((NODE_BLK, D), lambda i: (i, 0)),
            pl.BlockSpec((NODE_BLK, D), lambda i: (i, 0)),
        ],
        out_shape=[
            jax.ShapeDtypeStruct((N, D), jnp.float32),
            jax.ShapeDtypeStruct((N, D), jnp.float32),
        ],
    )(x, w_sd)


def _edge_body(e_ref, g_ref, g2_ref, w1_ref, w2_ref, w3_ref,
               b1_ref, b2_ref, b3_ref, o_ref):
    e = e_ref[...]
    h = _bdot(e, w1_ref[...]) + (g_ref[...] + g2_ref[...]) + b1_ref[...]
    h = jnp.maximum(_bdot(h, w2_ref[...]) + b2_ref[...], 0.0)
    h = _bdot(h, w3_ref[...]) + b3_ref[...]
    o_ref[...] = e + h


def _edge_mlp(e, gs, gd, w1a, w2, w3, b1, b2, b3):
    wspec = pl.BlockSpec((D, D), lambda i: (0, 0))
    bspec = pl.BlockSpec((1, D), lambda i: (0, 0))
    espec = pl.BlockSpec((EDGE_BLK, D), lambda i: (i, 0))
    return pl.pallas_call(
        _edge_body,
        grid=(E // EDGE_BLK,),
        in_specs=[espec, espec, espec, wspec, wspec, wspec, bspec, bspec, bspec],
        out_specs=espec,
        out_shape=jax.ShapeDtypeStruct((E, D), jnp.float32),
    )(e, gs, gd, w1a, w2, w3, b1, b2, b3)


def _node_body(x_ref, ps_ref, rec_ref, wn1_ref, wn2_ref, wn3_ref, wn4_ref,
               bn1_ref, bn2_ref, bn3_ref, bn4_ref, o_ref):
    x = x_ref[...]
    s = jnp.concatenate([ps_ref[0], ps_ref[1]], axis=1)
    m = s * rec_ref[:, :1]
    g = (_bdot(x, wn1_ref[:D, :]) + _bdot(m, wn1_ref[D:2 * D, :])
         + _bdot(s, wn1_ref[2 * D:, :]) + bn1_ref[...])
    g = jnp.maximum(g, 0.0)
    g = _bdot(g, wn2_ref[...]) + bn2_ref[...]
    g = jnp.maximum(_bdot(g, wn3_ref[...]) + bn3_ref[...], 0.0)
    g = jnp.maximum(_bdot(g, wn4_ref[...]) + bn4_ref[...], 0.0)
    o_ref[...] = x + g


def _node_mlp(x, pooled, rec16, wn1, wn2, wn3, wn4, bn1, bn2, bn3, bn4):
    return pl.pallas_call(
        _node_body,
        grid=(N // NODE_BLK,),
        in_specs=[
            pl.BlockSpec((NODE_BLK, D), lambda i: (i, 0)),
            pl.BlockSpec((2, NODE_BLK, H), lambda i: (0, i, 0)),
            pl.BlockSpec((NODE_BLK, 16), lambda i: (i, 0)),
            pl.BlockSpec((3 * D, 2 * D), lambda i: (0, 0)),
            pl.BlockSpec((2 * D, D), lambda i: (0, 0)),
            pl.BlockSpec((D, D), lambda i: (0, 0)),
            pl.BlockSpec((D, D), lambda i: (0, 0)),
            pl.BlockSpec((1, 2 * D), lambda i: (0, 0)),
            pl.BlockSpec((1, D), lambda i: (0, 0)),
            pl.BlockSpec((1, D), lambda i: (0, 0)),
            pl.BlockSpec((1, D), lambda i: (0, 0)),
        ],
        out_specs=pl.BlockSpec((NODE_BLK, D), lambda i: (i, 0)),
        out_shape=jax.ShapeDtypeStruct((N, D), jnp.float32),
    )(x, pooled, rec16, wn1, wn2, wn3, wn4, bn1, bn2, bn3, bn4)


def _gather_body(xs_hbm, xd_hbm, src_hbm, dst_hbm, gs_hbm, gd_hbm,
                 idx_s, idx_d, buf_s, buf_d, sem_s, sem_d):
    c = lax.axis_index("c")
    s = lax.axis_index("s")
    wid = s * 2 + c
    nfull = NCHUNK // 32
    nch = jnp.where(wid < NCHUNK - 32 * nfull, nfull + 1, nfull)

    def step(j, carry):
        base = (wid + 32 * j) * CH
        pltpu.sync_copy(src_hbm.at[pl.ds(base, CH)], idx_s)
        pltpu.sync_copy(dst_hbm.at[pl.ds(base, CH)], idx_d)
        cp_s = pltpu.async_copy(xs_hbm.at[idx_s], buf_s, sem_s)
        cp_d = pltpu.async_copy(xd_hbm.at[idx_d], buf_d, sem_d)
        cp_s.wait()
        pltpu.sync_copy(buf_s, gs_hbm.at[pl.ds(base, CH), :])
        cp_d.wait()
        pltpu.sync_copy(buf_d, gd_hbm.at[pl.ds(base, CH), :])
        return carry

    lax.fori_loop(0, nch, step, 0)


_sc_gather = functools.partial(
    pl.kernel,
    out_type=[
        jax.ShapeDtypeStruct((E, D), jnp.float32),
        jax.ShapeDtypeStruct((E, D), jnp.float32),
    ],
    mesh=_MESH,
    scratch_types=[
        pltpu.VMEM((CH,), jnp.int32),
        pltpu.VMEM((CH,), jnp.int32),
        pltpu.VMEM((CH, D), jnp.float32),
        pltpu.VMEM((CH, D), jnp.float32),
        pltpu.SemaphoreType.DMA,
        pltpu.SemaphoreType.DMA,
    ],
)(_gather_body)


def _zero_fill(ref, rows, width):
    def st(t, carry):
        r = t // (width // 16)
        q = t % (width // 16)
        ref[r, pl.ds(q * 16, 16)] = jnp.zeros((16,), jnp.float32)
        return carry
    lax.fori_loop(0, rows * (width // 16), st, 0)


def _scatter_common(e_hbm, dst_hbm, out_hbm, acc, idx_v, buf, sem,
                    cnt_extra=None):
    c = lax.axis_index("c")
    s = lax.axis_index("s")

    _zero_fill(buf, CH, H)
    for q in range(TROWS // CH):
        pltpu.sync_copy(buf, acc.at[pl.ds(s * TROWS + q * CH, CH)])
    if cnt_extra is not None:
        cacc, ones = cnt_extra

        @pl.when(c == 0)
        def _():
            _zero_fill(ones, CH, 16)
            for q in range(TROWS // CH):
                pltpu.sync_copy(ones,
                                cacc.at[pl.ds(s * TROWS + q * CH, CH)])

            def fill_ones(t, carry):
                ones[t, :] = jnp.full((16,), 1.0, jnp.float32)
                return carry
            lax.fori_loop(0, CH, fill_ones, 0)
    plsc.subcore_barrier()

    nfull = NCHUNK // 16
    nch = jnp.where(s < NCHUNK - 16 * nfull, nfull + 1, nfull)

    def step(j, carry):
        base = (s + 16 * j) * CH
        pltpu.sync_copy(dst_hbm.at[pl.ds(base, CH)], idx_v)

        @pl.when(c == 0)
        def _():
            pltpu.sync_copy(e_hbm.at[pl.ds(base, CH), pl.ds(0, H)], buf)

        @pl.when(c == 1)
        def _():
            pltpu.sync_copy(e_hbm.at[pl.ds(base, CH), pl.ds(H, H)], buf)

        pltpu.async_copy(buf, acc.at[idx_v], sem, add=True).wait()
        if cnt_extra is not None:
            cacc, ones = cnt_extra

            @pl.when(c == 0)
            def _():
                pltpu.async_copy(ones, cacc.at[idx_v], sem, add=True).wait()
        return carry

    lax.fori_loop(0, nch, step, 0)
    plsc.subcore_barrier()

    @pl.when(s < 15)
    def _():
        for q in range(TROWS // CH):
            r0 = s * TROWS + q * CH
            pltpu.sync_copy(acc.at[pl.ds(r0, CH)], buf)
            pltpu.sync_copy(buf, out_hbm.at[c, pl.ds(r0, CH), :])

    @pl.when(s == 15)
    def _():
        for q in range(3):
            r0 = 15 * TROWS + q * CH
            pltpu.sync_copy(acc.at[pl.ds(r0, CH)], buf)
            pltpu.sync_copy(buf, out_hbm.at[c, pl.ds(r0, CH), :])
        r0 = 15 * TROWS + 3 * CH
        pltpu.sync_copy(acc.at[pl.ds(r0, 16)], buf.at[pl.ds(0, 16)])
        pltpu.sync_copy(buf.at[pl.ds(0, 16)], out_hbm.at[c, pl.ds(r0, 16), :])


def _scatter_body(e_hbm, dst_hbm, out_hbm, acc, idx_v, buf, sem):
    _scatter_common(e_hbm, dst_hbm, out_hbm, acc, idx_v, buf, sem)


def _scatter_cnt_body(e_hbm, dst_hbm, out_hbm, cnt_hbm,
                      acc, cacc, idx_v, buf, ones, sem):
    c = lax.axis_index("c")
    s = lax.axis_index("s")
    _scatter_common(e_hbm, dst_hbm, out_hbm, acc, idx_v, buf, sem,
                    cnt_extra=(cacc, ones))

    @pl.when((c == 0) & (s < 15))
    def _():
        for q in range(TROWS // CH):
            r0 = s * TROWS + q * CH
            pltpu.sync_copy(cacc.at[pl.ds(r0, CH)], ones)
            pltpu.sync_copy(ones, cnt_hbm.at[pl.ds(r0, CH), :])

    @pl.when((c == 0) & (s == 15))
    def _():
        for q in range(3):
            r0 = 15 * TROWS + q * CH
            pltpu.sync_copy(cacc.at[pl.ds(r0, CH)], ones)
            pltpu.sync_copy(ones, cnt_hbm.at[pl.ds(r0, CH), :])
        r0 = 15 * TROWS + 3 * CH
        pltpu.sync_copy(cacc.at[pl.ds(r0, 16)], ones.at[pl.ds(0, 16)])
        pltpu.sync_copy(ones.at[pl.ds(0, 16)], cnt_hbm.at[pl.ds(r0, 16), :])


def _count_body(dst_hbm, out_hbm, acc, idx_v, buf, sem):
    c = lax.axis_index("c")
    s = lax.axis_index("s")

    _zero_fill(buf, CH, H)
    for q in range(TROWS // CH):
        pltpu.sync_copy(buf, acc.at[pl.ds(s * TROWS + q * CH, CH)])

    def fill_ones(t, carry):
        for q in range(H // 16):
            buf[t, pl.ds(q * 16, 16)] = jnp.full((16,), 1.0, jnp.float32)
        return carry
    lax.fori_loop(0, CH, fill_ones, 0)
    plsc.subcore_barrier()

    nfull = NCHUNK // 32
    wid = s * 2 + c
    nch = jnp.where(wid < NCHUNK - 32 * nfull, nfull + 1, nfull)

    def step(j, carry):
        base = (wid + 32 * j) * CH
        pltpu.sync_copy(dst_hbm.at[pl.ds(base, CH)], idx_v)
        pltpu.async_copy(buf, acc.at[idx_v], sem, add=True).wait()
        return carry

    lax.fori_loop(0, nch, step, 0)
    plsc.subcore_barrier()

    @pl.when(s < 15)
    def _():
        for q in range(TROWS // CH):
            r0 = s * TROWS + q * CH
            pltpu.sync_copy(acc.at[pl.ds(r0, CH)], buf)
            pltpu.sync_copy(buf, out_hbm.at[c, pl.ds(r0, CH), :])

    @pl.when(s == 15)
    def _():
        for q in range(3):
            r0 = 15 * TROWS + q * CH
            pltpu.sync_copy(acc.at[pl.ds(r0, CH)], buf)
            pltpu.sync_copy(buf, out_hbm.at[c, pl.ds(r0, CH), :])
        r0 = 15 * TROWS + 3 * CH
        pltpu.sync_copy(acc.at[pl.ds(r0, 16)], buf.at[pl.ds(0, 16)])
        pltpu.sync_copy(buf.at[pl.ds(0, 16)], out_hbm.at[c, pl.ds(r0, 16), :])


_sc_count = functools.partial(
    pl.kernel,
    out_type=jax.ShapeDtypeStruct((2, N, H), jnp.float32),
    mesh=_MESH,
    scratch_types=[
        pltpu.VMEM_SHARED((NPAD, H), jnp.float32),
        pltpu.VMEM((CH,), jnp.int32),
        pltpu.VMEM((CH, H), jnp.float32),
        pltpu.SemaphoreType.DMA,
    ],
)(_count_body)


_sc_scatter = functools.partial(
    pl.kernel,
    out_type=jax.ShapeDtypeStruct((2, N, H), jnp.float32),
    mesh=_MESH,
    scratch_types=[
        pltpu.VMEM_SHARED((NPAD, H), jnp.float32),
        pltpu.VMEM((CH,), jnp.int32),
        pltpu.VMEM((CH, H), jnp.float32),
        pltpu.SemaphoreType.DMA,
    ],
)(_scatter_body)


_sc_scatter_cnt = functools.partial(
    pl.kernel,
    out_type=[
        jax.ShapeDtypeStruct((2, N, H), jnp.float32),
        jax.ShapeDtypeStruct((N, 16), jnp.float32),
    ],
    mesh=_MESH,
    scratch_types=[
        pltpu.VMEM_SHARED((NPAD, H), jnp.float32),
        pltpu.VMEM_SHARED((NPAD, 16), jnp.float32),
        pltpu.VMEM((CH,), jnp.int32),
        pltpu.VMEM((CH, H), jnp.float32),
        pltpu.VMEM((CH, 16), jnp.float32),
        pltpu.SemaphoreType.DMA,
    ],
)(_scatter_cnt_body)


def kernel(x, edge_feat, edge_index, We1, be1, We2, be2, We3, be3,
           Wn1, bn1, Wn2, bn2, Wn3, bn3, Wn4, bn4):
    src32 = edge_index[0].astype(jnp.int32)
    dst32 = edge_index[1].astype(jnp.int32)

    w1a = We1[:D, :]
    w_sd = We1[D:, :].reshape(2, D, D)
    w_sd = jnp.concatenate([w_sd[0], w_sd[1]], axis=1)

    b1 = be1.reshape(1, D)
    b2 = be2.reshape(1, D)
    b3 = be3.reshape(1, D)
    nb1 = bn1.reshape(1, 2 * D)
    nb2 = bn2.reshape(1, D)
    nb3 = bn3.reshape(1, D)
    nb4 = bn4.reshape(1, D)

    e = edge_feat
    rec16 = None
    for it in range(3):
        xs, xd = _project(x, w_sd)
        gs, gd = _sc_gather(xs, xd, src32, dst32)
        e = _edge_mlp(e, gs, gd, w1a, We2, We3, b1, b2, b3)
        if it == 0:
            cnt2 = _sc_count(dst32)
            cnt16 = cnt2[0, :, :16] + cnt2[1, :, :16]
            rec16 = 1.0 / jnp.maximum(cnt16, 1.0)
        pooled = _sc_scatter(e, dst32)
        x = _node_mlp(x, pooled, rec16, Wn1, Wn2, Wn3, Wn4, nb1, nb2, nb3, nb4)

    return (x, e)


# speedup vs baseline: 2.8950x; 1.0325x over previous
import functools

import jax
import jax.numpy as jnp
from jax import lax
from jax.experimental import pallas as pl
from jax.experimental.pallas import tpu as pltpu
from jax.experimental.pallas import tpu_sc as plsc

N = 10000
E = 160000
D = 256
H = D // 2

EDGE_BLK = 2000
NODE_BLK = 1000

CH = 128
NCHUNK = E // CH
TROWS = 640
NPAD = 16 * TROWS

_MESH = plsc.VectorSubcoreMesh(core_axis_name="c", subcore_axis_name="s",
                               num_cores=2, num_subcores=16)


def _bdot(a, b):
    return jax.lax.dot(a.astype(jnp.bfloat16), b.astype(jnp.bfloat16),
                       preferred_element_type=jnp.float32)


def _proj_body(x_ref, w_ref, os_ref, od_ref):
    p = _bdot(x_ref[...], w_ref[...])
    os_ref[...] = p[:, :D]
    od_ref[...] = p[:, D:]


def _project(x, w_sd):
    return pl.pallas_call(
        _proj_body,
        grid=(N // NODE_BLK,),
        in_specs=[
            pl.BlockSpec((NODE_BLK, D), lambda i: (i, 0)),
            pl.BlockSpec((D, 2 * D), lambda i: (0, 0)),
        ],
        out_specs=[
            pl.BlockSpec((NODE_BLK, D), lambda i: (i, 0)),
            pl.BlockSpec((NODE_BLK, D), lambda i: (i, 0)),
        ],
        out_shape=[
            jax.ShapeDtypeStruct((N, D), jnp.float32),
            jax.ShapeDtypeStruct((N, D), jnp.float32),
        ],
    )(x, w_sd)


def _edge_body(e_ref, g_ref, g2_ref, w1_ref, w2_ref, w3_ref,
               b1_ref, b2_ref, b3_ref, o_ref):
    e = e_ref[...]
    h = _bdot(e, w1_ref[...]) + (g_ref[...] + g2_ref[...]) + b1_ref[...]
    h = jnp.maximum(_bdot(h, w2_ref[...]) + b2_ref[...], 0.0)
    h = _bdot(h, w3_ref[...]) + b3_ref[...]
    o_ref[...] = e + h


def _edge_mlp(e, gs, gd, w1a, w2, w3, b1, b2, b3):
    wspec = pl.BlockSpec((D, D), lambda i: (0, 0))
    bspec = pl.BlockSpec((1, D), lambda i: (0, 0))
    espec = pl.BlockSpec((EDGE_BLK, D), lambda i: (i, 0))
    return pl.pallas_call(
        _edge_body,
        grid=(E // EDGE_BLK,),
        in_specs=[espec, espec, espec, wspec, wspec, wspec, bspec, bspec, bspec],
        out_specs=espec,
        out_shape=jax.ShapeDtypeStruct((E, D), jnp.float32),
    )(e, gs, gd, w1a, w2, w3, b1, b2, b3)


def _node_body(x_ref, ps_ref, rec_ref, wn1_ref, wn2_ref, wn3_ref, wn4_ref,
               bn1_ref, bn2_ref, bn3_ref, bn4_ref, o_ref):
    x = x_ref[...]
    s = jnp.concatenate([ps_ref[0], ps_ref[1]], axis=1)
    m = s * rec_ref[:, :1]
    g = (_bdot(x, wn1_ref[:D, :]) + _bdot(m, wn1_ref[D:2 * D, :])
         + _bdot(s, wn1_ref[2 * D:, :]) + bn1_ref[...])
    g = jnp.maximum(g, 0.0)
    g = _bdot(g, wn2_ref[...]) + bn2_ref[...]
    g = jnp.maximum(_bdot(g, wn3_ref[...]) + bn3_ref[...], 0.0)
    g = jnp.maximum(_bdot(g, wn4_ref[...]) + bn4_ref[...], 0.0)
    o_ref[...] = x + g


def _node_mlp(x, pooled, rec16, wn1, wn2, wn3, wn4, bn1, bn2, bn3, bn4):
    return pl.pallas_call(
        _node_body,
        grid=(N // NODE_BLK,),
        in_specs=[
            pl.BlockSpec((NODE_BLK, D), lambda i: (i, 0)),
            pl.BlockSpec((2, NODE_BLK, H), lambda i: (0, i, 0)),
            pl.BlockSpec((NODE_BLK, 16), lambda i: (i, 0)),
            pl.BlockSpec((3 * D, 2 * D), lambda i: (0, 0)),
            pl.BlockSpec((2 * D, D), lambda i: (0, 0)),
            pl.BlockSpec((D, D), lambda i: (0, 0)),
            pl.BlockSpec((D, D), lambda i: (0, 0)),
            pl.BlockSpec((1, 2 * D), lambda i: (0, 0)),
            pl.BlockSpec((1, D), lambda i: (0, 0)),
            pl.BlockSpec((1, D), lambda i: (0, 0)),
            pl.BlockSpec((1, D), lambda i: (0, 0)),
        ],
        out_specs=pl.BlockSpec((NODE_BLK, D), lambda i: (i, 0)),
        out_shape=jax.ShapeDtypeStruct((N, D), jnp.float32),
    )(x, pooled, rec16, wn1, wn2, wn3, wn4, bn1, bn2, bn3, bn4)


def _gather_body(xs_hbm, xd_hbm, src_hbm, dst_hbm, gs_hbm, gd_hbm,
                 idx_s, idx_d, buf_s, buf_d, sem_s, sem_d):
    c = lax.axis_index("c")
    s = lax.axis_index("s")
    wid = s * 2 + c
    nfull = NCHUNK // 32
    nch = jnp.where(wid < NCHUNK - 32 * nfull, nfull + 1, nfull)

    def step(j, carry):
        base = (wid + 32 * j) * CH
        pltpu.sync_copy(src_hbm.at[pl.ds(base, CH)], idx_s)
        pltpu.sync_copy(dst_hbm.at[pl.ds(base, CH)], idx_d)
        cp_s = pltpu.async_copy(xs_hbm.at[idx_s], buf_s, sem_s)
        cp_d = pltpu.async_copy(xd_hbm.at[idx_d], buf_d, sem_d)
        cp_s.wait()
        pltpu.sync_copy(buf_s, gs_hbm.at[pl.ds(base, CH), :])
        cp_d.wait()
        pltpu.sync_copy(buf_d, gd_hbm.at[pl.ds(base, CH), :])
        return carry

    lax.fori_loop(0, nch, step, 0)


_sc_gather = functools.partial(
    pl.kernel,
    out_type=[
        jax.ShapeDtypeStruct((E, D), jnp.float32),
        jax.ShapeDtypeStruct((E, D), jnp.float32),
    ],
    mesh=_MESH,
    scratch_types=[
        pltpu.VMEM((CH,), jnp.int32),
        pltpu.VMEM((CH,), jnp.int32),
        pltpu.VMEM((CH, D), jnp.float32),
        pltpu.VMEM((CH, D), jnp.float32),
        pltpu.SemaphoreType.DMA,
        pltpu.SemaphoreType.DMA,
    ],
)(_gather_body)


def _zero_fill(ref, rows, width):
    def st(t, carry):
        r = t // (width // 16)
        q = t % (width // 16)
        ref[r, pl.ds(q * 16, 16)] = jnp.zeros((16,), jnp.float32)
        return carry
    lax.fori_loop(0, rows * (width // 16), st, 0)


def _scatter_common(e_hbm, dst_hbm, out_hbm, acc, idx_v, buf, idx_b, buf_b,
                    sem, sem_b):
    c = lax.axis_index("c")
    s = lax.axis_index("s")

    _zero_fill(buf, CH, H)
    for q in range(TROWS // CH):
        pltpu.sync_copy(buf, acc.at[pl.ds(s * TROWS + q * CH, CH)])
    plsc.subcore_barrier()

    nfull = NCHUNK // 16
    nch = jnp.where(s < NCHUNK - 16 * nfull, nfull + 1, nfull)

    def load(j, ib, bb):
        base = (s + 16 * j) * CH
        pltpu.sync_copy(dst_hbm.at[pl.ds(base, CH)], ib)

        @pl.when(c == 0)
        def _():
            pltpu.sync_copy(e_hbm.at[pl.ds(base, CH), pl.ds(0, H)], bb)

        @pl.when(c == 1)
        def _():
            pltpu.sync_copy(e_hbm.at[pl.ds(base, CH), pl.ds(H, H)], bb)

    def pair(jj, carry):
        load(2 * jj, idx_v, buf)
        cp_a = pltpu.async_copy(buf, acc.at[idx_v], sem, add=True)
        load(2 * jj + 1, idx_b, buf_b)
        cp_b = pltpu.async_copy(buf_b, acc.at[idx_b], sem_b, add=True)
        cp_a.wait()
        cp_b.wait()
        return carry

    lax.fori_loop(0, nch // 2, pair, 0)

    @pl.when(nch % 2 == 1)
    def _():
        load(nch - 1, idx_v, buf)
        pltpu.async_copy(buf, acc.at[idx_v], sem, add=True).wait()
    plsc.subcore_barrier()

    @pl.when(s < 15)
    def _():
        for q in range(TROWS // CH):
            r0 = s * TROWS + q * CH
            pltpu.sync_copy(acc.at[pl.ds(r0, CH)], buf)
            pltpu.sync_copy(buf, out_hbm.at[c, pl.ds(r0, CH), :])

    @pl.when(s == 15)
    def _():
        for q in range(3):
            r0 = 15 * TROWS + q * CH
            pltpu.sync_copy(acc.at[pl.ds(r0, CH)], buf)
            pltpu.sync_copy(buf, out_hbm.at[c, pl.ds(r0, CH), :])
        r0 = 15 * TROWS + 3 * CH
        pltpu.sync_copy(acc.at[pl.ds(r0, 16)], buf.at[pl.ds(0, 16)])
        pltpu.sync_copy(buf.at[pl.ds(0, 16)], out_hbm.at[c, pl.ds(r0, 16), :])


def _scatter_body(e_hbm, dst_hbm, out_hbm, acc, idx_v, buf, idx_b, buf_b,
                  sem, sem_b):
    _scatter_common(e_hbm, dst_hbm, out_hbm, acc, idx_v, buf, idx_b, buf_b,
                    sem, sem_b)


def _count_body(dst_hbm, out_hbm, acc, idx_v, buf, sem):
    c = lax.axis_index("c")
    s = lax.axis_index("s")

    _zero_fill(buf, CH, H)
    for q in range(TROWS // CH):
        pltpu.sync_copy(buf, acc.at[pl.ds(s * TROWS + q * CH, CH)])

    def fill_ones(t, carry):
        for q in range(H // 16):
            buf[t, pl.ds(q * 16, 16)] = jnp.full((16,), 1.0, jnp.float32)
        return carry
    lax.fori_loop(0, CH, fill_ones, 0)
    plsc.subcore_barrier()

    nfull = NCHUNK // 32
    wid = s * 2 + c
    nch = jnp.where(wid < NCHUNK - 32 * nfull, nfull + 1, nfull)

    def step(j, carry):
        base = (wid + 32 * j) * CH
        pltpu.sync_copy(dst_hbm.at[pl.ds(base, CH)], idx_v)
        pltpu.async_copy(buf, acc.at[idx_v], sem, add=True).wait()
        return carry

    lax.fori_loop(0, nch, step, 0)
    plsc.subcore_barrier()

    @pl.when(s < 15)
    def _():
        for q in range(TROWS // CH):
            r0 = s * TROWS + q * CH
            pltpu.sync_copy(acc.at[pl.ds(r0, CH)], buf)
            pltpu.sync_copy(buf, out_hbm.at[c, pl.ds(r0, CH), :])

    @pl.when(s == 15)
    def _():
        for q in range(3):
            r0 = 15 * TROWS + q * CH
            pltpu.sync_copy(acc.at[pl.ds(r0, CH)], buf)
            pltpu.sync_copy(buf, out_hbm.at[c, pl.ds(r0, CH), :])
        r0 = 15 * TROWS + 3 * CH
        pltpu.sync_copy(acc.at[pl.ds(r0, 16)], buf.at[pl.ds(0, 16)])
        pltpu.sync_copy(buf.at[pl.ds(0, 16)], out_hbm.at[c, pl.ds(r0, 16), :])


_sc_count = functools.partial(
    pl.kernel,
    out_type=jax.ShapeDtypeStruct((2, N, H), jnp.float32),
    mesh=_MESH,
    scratch_types=[
        pltpu.VMEM_SHARED((NPAD, H), jnp.float32),
        pltpu.VMEM((CH,), jnp.int32),
        pltpu.VMEM((CH, H), jnp.float32),
        pltpu.SemaphoreType.DMA,
    ],
)(_count_body)


_sc_scatter = functools.partial(
    pl.kernel,
    out_type=jax.ShapeDtypeStruct((2, N, H), jnp.float32),
    mesh=_MESH,
    scratch_types=[
        pltpu.VMEM_SHARED((NPAD, H), jnp.float32),
        pltpu.VMEM((CH,), jnp.int32),
        pltpu.VMEM((CH, H), jnp.float32),
        pltpu.VMEM((CH,), jnp.int32),
        pltpu.VMEM((CH, H), jnp.float32),
        pltpu.SemaphoreType.DMA,
        pltpu.SemaphoreType.DMA,
    ],
)(_scatter_body)


def kernel(x, edge_feat, edge_index, We1, be1, We2, be2, We3, be3,
           Wn1, bn1, Wn2, bn2, Wn3, bn3, Wn4, bn4):
    src32 = edge_index[0].astype(jnp.int32)
    dst32 = edge_index[1].astype(jnp.int32)

    w1a = We1[:D, :]
    w_sd = We1[D:, :].reshape(2, D, D)
    w_sd = jnp.concatenate([w_sd[0], w_sd[1]], axis=1)

    b1 = be1.reshape(1, D)
    b2 = be2.reshape(1, D)
    b3 = be3.reshape(1, D)
    nb1 = bn1.reshape(1, 2 * D)
    nb2 = bn2.reshape(1, D)
    nb3 = bn3.reshape(1, D)
    nb4 = bn4.reshape(1, D)

    e = edge_feat
    rec16 = None
    for it in range(3):
        xs, xd = _project(x, w_sd)
        gs, gd = _sc_gather(xs, xd, src32, dst32)
        e = _edge_mlp(e, gs, gd, w1a, We2, We3, b1, b2, b3)
        if it == 0:
            cnt2 = _sc_count(dst32)
            cnt16 = cnt2[0, :, :16] + cnt2[1, :, :16]
            rec16 = 1.0 / jnp.maximum(cnt16, 1.0)
        pooled = _sc_scatter(e, dst32)
        x = _node_mlp(x, pooled, rec16, Wn1, Wn2, Wn3, Wn4, nb1, nb2, nb3, nb4)

    return (x, e)
